# baseline (speedup 1.0000x reference)
import functools

import jax
import jax.numpy as jnp
from jax import lax
from jax.experimental import pallas as pl
from jax.experimental.pallas import tpu as pltpu
from jax.experimental.pallas import tpu_sc as plsc

_NC = 2
_NS = 16
_NW = _NC * _NS

_B = 16384
_D = 32
_BPW = _B // _NW
_CH = 128
_NCHK = _BPW // _CH
_W = 128


def _fetch_rows(tab1d, idx_v, chunk_base, fat, sem):
    copies = []
    for k in range(_CH // 16):
        vec = idx_v[pl.ds(chunk_base + k * 16, 16)]
        for l in range(16):
            off = pl.multiple_of(
                jnp.bitwise_and(vec[l], -4) * _D, _W)
            copies.append(
                pltpu.async_copy(tab1d.at[pl.ds(off, _W)],
                                 fat.at[k * 16 + l], sem))
    return copies


def _sc_gather_body(uidx_hbm, iidx_hbm, gu_hbm, gi_hbm, mu_hbm, mi_hbm,
                    out_gmf, out_mu, out_mi,
                    u_v, i_v, fat_a, fat_b, ost,
                    isem, gsem_a, gsem_b, wsem):
    wid = lax.axis_index("s") * _NC + lax.axis_index("c")
    base = wid * _BPW

    ic1 = pltpu.async_copy(uidx_hbm.at[pl.ds(base, _BPW)], u_v, isem)
    ic2 = pltpu.async_copy(iidx_hbm.at[pl.ds(base, _BPW)], i_v, isem)
    ic1.wait()
    ic2.wait()

    def extract(fat, idx_v, chunk_base, k, other=None, idx2_v=None):
        jv = lax.iota(jnp.int32, 16) + k * 16
        sub = jnp.bitwise_and(idx_v[pl.ds(chunk_base + k * 16, 16)], 3) * _D
        if other is not None:
            sub2 = jnp.bitwise_and(
                idx2_v[pl.ds(chunk_base + k * 16, 16)], 3) * _D
        dst = jv * _D
        for col in range(_D):
            v = plsc.load_gather(fat, [jv, sub + col])
            if other is not None:
                v = v * plsc.load_gather(other, [jv, sub2 + col])
            plsc.store_scatter(ost, [dst + col], v)

    def gmf_chunk(c, carry):
        ca = _fetch_rows(gu_hbm, u_v, c * _CH, fat_a, gsem_a)
        cb = _fetch_rows(gi_hbm, i_v, c * _CH, fat_b, gsem_b)
        for d in ca:
            d.wait()
        for d in cb:
            d.wait()
        for k in range(_CH // 16):
            extract(fat_a, u_v, c * _CH, k, other=fat_b, idx2_v=i_v)
        pltpu.async_copy(ost, out_gmf.at[pl.ds((base + c * _CH) * _D,
                                               _CH * _D)], wsem).wait()
        return carry

    def mlp_chunk(idx_v, tab, out):
        def body(c, carry):
            for d in _fetch_rows(tab, idx_v, c * _CH, fat_a, gsem_a):
                d.wait()
            for k in range(_CH // 16):
                extract(fat_a, idx_v, c * _CH, k)
            pltpu.async_copy(ost, out.at[pl.ds((base + c * _CH) * _D,
                                               _CH * _D)], wsem).wait()
            return carry
        return body

    lax.fori_loop(0, _NCHK, gmf_chunk, 0)
    lax.fori_loop(0, _NCHK, mlp_chunk(u_v, mu_hbm, out_mu), 0)
    lax.fori_loop(0, _NCHK, mlp_chunk(i_v, mi_hbm, out_mi), 0)


@functools.lru_cache(maxsize=None)
def _make_sc_gather():
    return pl.kernel(
        _sc_gather_body,
        out_type=[jax.ShapeDtypeStruct((_B * _D,), jnp.float32)] * 3,
        mesh=plsc.VectorSubcoreMesh(core_axis_name="c", subcore_axis_name="s",
                                    num_cores=_NC, num_subcores=_NS),
        scratch_types=[
            pltpu.VMEM((_BPW,), jnp.int32),
            pltpu.VMEM((_BPW,), jnp.int32),
            pltpu.VMEM((_CH, _W), jnp.float32),
            pltpu.VMEM((_CH, _W), jnp.float32),
            pltpu.VMEM((_CH * _D,), jnp.float32),
            pltpu.SemaphoreType.DMA,
            pltpu.SemaphoreType.DMA,
            pltpu.SemaphoreType.DMA,
            pltpu.SemaphoreType.DMA,
        ],
        compiler_params=pltpu.CompilerParams(needs_layout_passes=False),
    )


_BLK = 2048


def _tc_head_body(gmf_ref, mu_ref, mi_ref,
                  w1u_ref, w1i_ref, b1_ref, w2_ref, b2_ref, w3_ref, b3_ref,
                  wpg_ref, wph_ref, bp_ref, out_ref):
    h = jnp.maximum(
        jnp.dot(mu_ref[...], w1u_ref[...], preferred_element_type=jnp.float32)
        + jnp.dot(mi_ref[...], w1i_ref[...], preferred_element_type=jnp.float32)
        + b1_ref[...], 0.0)
    h = jnp.maximum(
        jnp.dot(h, w2_ref[...], preferred_element_type=jnp.float32)
        + b2_ref[...], 0.0)
    h = jnp.maximum(
        jnp.dot(h, w3_ref[...], preferred_element_type=jnp.float32)
        + b3_ref[...], 0.0)
    logit = (jnp.dot(gmf_ref[...], wpg_ref[...],
                     preferred_element_type=jnp.float32)
             + jnp.dot(h, wph_ref[...], preferred_element_type=jnp.float32)
             + bp_ref[...])
    out_ref[...] = 1.0 / (1.0 + jnp.exp(-logit))


def kernel(user_indices, item_indices, gmf_user_emb, gmf_item_emb,
           mlp_user_emb, mlp_item_emb, W1, b1, W2, b2, W3, b3, Wp, bp):
    uidx = user_indices.astype(jnp.int32)
    iidx = item_indices.astype(jnp.int32)

    flat = lambda t: t.reshape(-1)

    gmf1, mu1, mi1 = _make_sc_gather()(
        uidx, iidx, flat(gmf_user_emb), flat(gmf_item_emb),
        flat(mlp_user_emb), flat(mlp_item_emb))

    gmf = gmf1.reshape(_B, _D)
    mu = mu1.reshape(_B, _D)
    mi = mi1.reshape(_B, _D)

    n_blk = _B // _BLK
    row_spec = pl.BlockSpec((_BLK, _D), lambda i: (i, 0))
    full = lambda shape: pl.BlockSpec(shape, lambda i: (0,) * len(shape))

    out = pl.pallas_call(
        _tc_head_body,
        grid=(n_blk,),
        in_specs=[
            row_spec, row_spec, row_spec,
            full((_D, 64)), full((_D, 64)), full((1, 64)),
            full((64, 32)), full((1, 32)),
            full((32, 16)), full((1, 16)),
            full((_D, 1)), full((16, 1)), full((1, 1)),
        ],
        out_specs=pl.BlockSpec((_BLK, 1), lambda i: (i, 0)),
        out_shape=jax.ShapeDtypeStruct((_B, 1), jnp.float32),
    )(gmf, mu, mi,
      W1[:_D], W1[_D:], b1.reshape(1, 64),
      W2, b2.reshape(1, 32),
      W3, b3.reshape(1, 16),
      Wp[:_D], Wp[_D:], bp.reshape(1, 1))

    return out.reshape(-1)


# speedup vs baseline: 2.4596x; 2.4596x over previous
import functools

import jax
import jax.numpy as jnp
from jax import lax
from jax.experimental import pallas as pl
from jax.experimental.pallas import tpu as pltpu
from jax.experimental.pallas import tpu_sc as plsc

_NC = 2
_NS = 16
_NW = _NC * _NS

_B = 16384
_D = 32
_BPW = _B // _NW
_CH = 32
_NCHK = _BPW // _CH
_G = 32


def _fetch_rows(tab, idx_v, chunk_base, stage, sem):
    copies = []
    for k in range(_CH // 16):
        vec = idx_v[pl.ds(chunk_base + k * 16, 16)]
        for l in range(16):
            v = vec[l]
            g = jnp.right_shift(v, 5)
            r = jnp.bitwise_and(v, _G - 1)
            copies.append(
                pltpu.async_copy(tab.at[g, r], stage.at[k * 16 + l], sem))
    return copies


def _sc_gather_body(uidx_hbm, iidx_hbm, gu_hbm, gi_hbm, mu_hbm, mi_hbm,
                    out_gmf, out_mu, out_mi,
                    u_v, i_v, stage_a, stage_b,
                    isem, gsem_a, gsem_b, wsem):
    wid = lax.axis_index("s") * _NC + lax.axis_index("c")
    base = wid * _BPW

    ic1 = pltpu.async_copy(uidx_hbm.at[pl.ds(base, _BPW)], u_v, isem)
    ic2 = pltpu.async_copy(iidx_hbm.at[pl.ds(base, _BPW)], i_v, isem)
    ic1.wait()
    ic2.wait()

    def gmf_chunk(c, carry):
        ca = _fetch_rows(gu_hbm, u_v, c * _CH, stage_a, gsem_a)
        cb = _fetch_rows(gi_hbm, i_v, c * _CH, stage_b, gsem_b)
        for d in ca:
            d.wait()
        for d in cb:
            d.wait()
        for j in range(_CH):
            for half in range(_D // 16):
                o = pl.ds(half * 16, 16)
                stage_a[j, o] = stage_a[j, o] * stage_b[j, o]
        pltpu.async_copy(stage_a, out_gmf.at[pl.ds(base + c * _CH, _CH)],
                         wsem).wait()
        return carry

    def mlp_chunk(idx_v, tab, out):
        def body(c, carry):
            for d in _fetch_rows(tab, idx_v, c * _CH, stage_a, gsem_a):
                d.wait()
            pltpu.async_copy(stage_a, out.at[pl.ds(base + c * _CH, _CH)],
                             wsem).wait()
            return carry
        return body

    lax.fori_loop(0, _NCHK, gmf_chunk, 0)
    lax.fori_loop(0, _NCHK, mlp_chunk(u_v, mu_hbm, out_mu), 0)
    lax.fori_loop(0, _NCHK, mlp_chunk(i_v, mi_hbm, out_mi), 0)


@functools.lru_cache(maxsize=None)
def _make_sc_gather():
    return pl.kernel(
        _sc_gather_body,
        out_type=[jax.ShapeDtypeStruct((_B, _D), jnp.float32)] * 3,
        mesh=plsc.VectorSubcoreMesh(core_axis_name="c", subcore_axis_name="s",
                                    num_cores=_NC, num_subcores=_NS),
        scratch_types=[
            pltpu.VMEM((_BPW,), jnp.int32),
            pltpu.VMEM((_BPW,), jnp.int32),
            pltpu.VMEM((_CH, _D), jnp.float32),
            pltpu.VMEM((_CH, _D), jnp.float32),
            pltpu.SemaphoreType.DMA,
            pltpu.SemaphoreType.DMA,
            pltpu.SemaphoreType.DMA,
            pltpu.SemaphoreType.DMA,
        ],
        compiler_params=pltpu.CompilerParams(needs_layout_passes=False),
    )


_BLK = 2048


def _tc_head_body(gmf_ref, mu_ref, mi_ref,
                  w1u_ref, w1i_ref, b1_ref, w2_ref, b2_ref, w3_ref, b3_ref,
                  wpg_ref, wph_ref, bp_ref, out_ref):
    h = jnp.maximum(
        jnp.dot(mu_ref[...], w1u_ref[...], preferred_element_type=jnp.float32)
        + jnp.dot(mi_ref[...], w1i_ref[...], preferred_element_type=jnp.float32)
        + b1_ref[...], 0.0)
    h = jnp.maximum(
        jnp.dot(h, w2_ref[...], preferred_element_type=jnp.float32)
        + b2_ref[...], 0.0)
    h = jnp.maximum(
        jnp.dot(h, w3_ref[...], preferred_element_type=jnp.float32)
        + b3_ref[...], 0.0)
    logit = (jnp.dot(gmf_ref[...], wpg_ref[...],
                     preferred_element_type=jnp.float32)
             + jnp.dot(h, wph_ref[...], preferred_element_type=jnp.float32)
             + bp_ref[...])
    out_ref[...] = 1.0 / (1.0 + jnp.exp(-logit))


def kernel(user_indices, item_indices, gmf_user_emb, gmf_item_emb,
           mlp_user_emb, mlp_item_emb, W1, b1, W2, b2, W3, b3, Wp, bp):
    uidx = user_indices.astype(jnp.int32)
    iidx = item_indices.astype(jnp.int32)

    as3d = lambda t: t.reshape(t.shape[0] // _G, _G, _D)

    gmf, mu, mi = _make_sc_gather()(
        uidx, iidx, as3d(gmf_user_emb), as3d(gmf_item_emb),
        as3d(mlp_user_emb), as3d(mlp_item_emb))

    n_blk = _B // _BLK
    row_spec = pl.BlockSpec((_BLK, _D), lambda i: (i, 0))
    full = lambda shape: pl.BlockSpec(shape, lambda i: (0,) * len(shape))

    out = pl.pallas_call(
        _tc_head_body,
        grid=(n_blk,),
        in_specs=[
            row_spec, row_spec, row_spec,
            full((_D, 64)), full((_D, 64)), full((1, 64)),
            full((64, 32)), full((1, 32)),
            full((32, 16)), full((1, 16)),
            full((_D, 1)), full((16, 1)), full((1, 1)),
        ],
        out_specs=pl.BlockSpec((_BLK, 1), lambda i: (i, 0)),
        out_shape=jax.ShapeDtypeStruct((_B, 1), jnp.float32),
    )(gmf, mu, mi,
      W1[:_D], W1[_D:], b1.reshape(1, 64),
      W2, b2.reshape(1, 32),
      W3, b3.reshape(1, 16),
      Wp[:_D], Wp[_D:], bp.reshape(1, 1))

    return out.reshape(-1)
